# Initial kernel scaffold; baseline (speedup 1.0000x reference)
import jax
import jax.numpy as jnp
from jax.experimental import pallas as pl


def kernel(inputs, gate_w, w1, w3, w2):
    raise NotImplementedError("write your pallas kernel here")


# speedup vs baseline: 1.0467x; 1.0467x over previous
import functools

import jax
import jax.numpy as jnp
import numpy as np
from jax.experimental import pallas as pl
from jax.experimental.pallas import tpu as pltpu

B, H, E, F = 32, 2048, 8, 4096
BF = 512
NF = F // BF

_MASK_VAL = float(np.finfo(np.float32).min)


def _moe_kernel(x_ref, gw_ref, w1_ref, w3_ref, w2_ref, out_ref, scale_ref):
    e = pl.program_id(0)
    j = pl.program_id(1)

    @pl.when(j == 0)
    def _gate():
        x = x_ref[...]
        logits = jnp.dot(x, gw_ref[...], preferred_element_type=jnp.float32)
        m0 = jnp.max(logits, axis=1, keepdims=True)
        cond0 = logits == m0
        masked = jnp.where(cond0, _MASK_VAL, logits)
        m1 = jnp.max(masked, axis=1, keepdims=True)
        cond1 = logits == m1
        pre = 1.0 / (1.0 + jnp.exp(m1 - m0))
        w_all = (cond0.astype(jnp.float32) * pre
                 - cond1.astype(jnp.float32) * (pre - 1.0))
        onehot = jax.lax.broadcasted_iota(jnp.int32, (1, E), 1) == e
        scale_ref[...] = jnp.sum(jnp.where(onehot, w_all, 0.0), axis=1,
                                 keepdims=True)

    @pl.when((e == 0) & (j == 0))
    def _zero():
        out_ref[...] = jnp.zeros_like(out_ref)

    x = x_ref[...]
    h1 = jnp.dot(x, w1_ref[0], preferred_element_type=jnp.float32)
    h3 = jnp.dot(x, w3_ref[0], preferred_element_type=jnp.float32)
    hidden = (h1 * jax.nn.sigmoid(h1)) * h3
    hidden = hidden * scale_ref[...]
    out_ref[...] += jnp.dot(hidden, w2_ref[0], preferred_element_type=jnp.float32)


@functools.partial(jax.jit, static_argnames=("interpret",))
def _moe(x, gate_w, w1, w3, w2, interpret=False):
    return pl.pallas_call(
        _moe_kernel,
        grid=(E, NF),
        in_specs=[
            pl.BlockSpec((B, H), lambda e, j: (0, 0)),
            pl.BlockSpec((H, E), lambda e, j: (0, 0)),
            pl.BlockSpec((1, H, BF), lambda e, j: (e, 0, j)),
            pl.BlockSpec((1, H, BF), lambda e, j: (e, 0, j)),
            pl.BlockSpec((1, BF, H), lambda e, j: (e, j, 0)),
        ],
        out_specs=pl.BlockSpec((B, H), lambda e, j: (0, 0)),
        out_shape=jax.ShapeDtypeStruct((B, H), jnp.float32),
        scratch_shapes=[pltpu.VMEM((B, 1), jnp.float32)],
        compiler_params=pltpu.CompilerParams(
            dimension_semantics=("arbitrary", "arbitrary"),
        ),
        interpret=interpret,
    )(x, gate_w, w1, w3, w2)


def kernel(inputs, gate_w, w1, w3, w2):
    x = inputs.reshape(B, H)
    out = _moe(x, gate_w, w1, w3, w2)
    return out.reshape(1, 1, B, H)


# speedup vs baseline: 1.0717x; 1.0239x over previous
import functools

import jax
import jax.numpy as jnp
import numpy as np
from jax.experimental import pallas as pl
from jax.experimental.pallas import tpu as pltpu

B, H, E, F = 32, 2048, 8, 4096
BF = 512
NF = F // BF

_MASK_VAL = float(np.finfo(np.float32).min)


def _moe_kernel(x_ref, gw_ref, w1_ref, w3_ref, w2_ref, out_ref, scale_ref):
    e = pl.program_id(0)
    j = pl.program_id(1)

    @pl.when(j == 0)
    def _gate():
        x = x_ref[...]
        logits = jnp.dot(x, gw_ref[...], preferred_element_type=jnp.float32)
        m0 = jnp.max(logits, axis=1, keepdims=True)
        cond0 = logits == m0
        masked = jnp.where(cond0, _MASK_VAL, logits)
        m1 = jnp.max(masked, axis=1, keepdims=True)
        cond1 = logits == m1
        pre = 1.0 / (1.0 + jnp.exp(m1 - m0))
        w_all = (cond0.astype(jnp.float32) * pre
                 - cond1.astype(jnp.float32) * (pre - 1.0))
        onehot = jax.lax.broadcasted_iota(jnp.int32, (1, E), 1) == e
        scale_ref[...] = jnp.sum(jnp.where(onehot, w_all, 0.0), axis=1,
                                 keepdims=True)

    @pl.when((e == 0) & (j == 0))
    def _zero():
        out_ref[...] = jnp.zeros_like(out_ref)

    xb = x_ref[...].astype(jnp.bfloat16)
    h1 = jnp.dot(xb, w1_ref[0].astype(jnp.bfloat16),
                 preferred_element_type=jnp.float32)
    h3 = jnp.dot(xb, w3_ref[0].astype(jnp.bfloat16),
                 preferred_element_type=jnp.float32)
    hidden = (h1 * jax.nn.sigmoid(h1)) * h3
    hidden = hidden * scale_ref[...]
    out_ref[...] += jnp.dot(hidden.astype(jnp.bfloat16),
                            w2_ref[0].astype(jnp.bfloat16),
                            preferred_element_type=jnp.float32)


@functools.partial(jax.jit, static_argnames=("interpret",))
def _moe(x, gate_w, w1, w3, w2, interpret=False):
    return pl.pallas_call(
        _moe_kernel,
        grid=(E, NF),
        in_specs=[
            pl.BlockSpec((B, H), lambda e, j: (0, 0)),
            pl.BlockSpec((H, E), lambda e, j: (0, 0)),
            pl.BlockSpec((1, H, BF), lambda e, j: (e, 0, j)),
            pl.BlockSpec((1, H, BF), lambda e, j: (e, 0, j)),
            pl.BlockSpec((1, BF, H), lambda e, j: (e, j, 0)),
        ],
        out_specs=pl.BlockSpec((B, H), lambda e, j: (0, 0)),
        out_shape=jax.ShapeDtypeStruct((B, H), jnp.float32),
        scratch_shapes=[pltpu.VMEM((B, 1), jnp.float32)],
        compiler_params=pltpu.CompilerParams(
            dimension_semantics=("arbitrary", "arbitrary"),
        ),
        interpret=interpret,
    )(x, gate_w, w1, w3, w2)


def kernel(inputs, gate_w, w1, w3, w2):
    x = inputs.reshape(B, H)
    out = _moe(x, gate_w, w1, w3, w2)
    return out.reshape(1, 1, B, H)
